# baseline (speedup 1.0000x reference)
import functools

import jax
import jax.numpy as jnp
from jax import lax
from jax.experimental import pallas as pl
from jax.experimental.pallas import tpu as pltpu
from jax.experimental.pallas import tpu_sc as plsc

_EPS = 1e-5
_N, _V, _D = 16384, 100000, 128
_NC, _NS, _L = 2, 16, 16
_NW = _NC * _NS
_CHUNK = _N // _NW
_GCH = 128
_NG = _CHUNK // _GCH
_SLICE = _N // _NS


def _sc_body(values_hbm, numbers_hbm, table_hbm, gamma_hbm, beta_hbm, out_hbm,
             idx_v, nums_v, rows_v, norm_v, flags_v, gb_v, pack_v, all_v,
             shared, gsems, osem):
    cid = lax.axis_index("c")
    sid = lax.axis_index("s")
    wid = sid * _NC + cid
    base = wid * _CHUNK

    with jax.named_scope("stage"):
        pltpu.sync_copy(values_hbm.at[pl.ds(wid * _NG, _NG)], idx_v)
        pltpu.sync_copy(numbers_hbm.at[pl.ds(sid * _SLICE, _SLICE)], nums_v)
        pltpu.sync_copy(gamma_hbm, gb_v.at[pl.ds(0, 1)])
        pltpu.sync_copy(beta_hbm, gb_v.at[pl.ds(8, 1)])
    with jax.named_scope("gfire"):
        gathers = [
            pltpu.async_copy(table_hbm.at[idx_v.at[j]],
                             rows_v.at[pl.ds(j * _GCH, _GCH)], gsems.at[j])
            for j in range(_NG)
        ]

    def stats_step(i, carry):
        s, ss, cnt = carry
        for u in range(8):
            x = nums_v[pl.ds(i * 8 * _L + u * _L, _L)]
            pres = x == x
            xs = jnp.where(pres, x, 0.0)
            s = s + xs
            ss = ss + xs * xs
            cnt = cnt + jnp.where(pres, 1.0, 0.0)
        return s, ss, cnt

    with jax.named_scope("stats"):
        zero = jnp.zeros((_L,), jnp.float32)
        s, ss, cnt = lax.fori_loop(0, _SLICE // (8 * _L), stats_step,
                                   (zero, zero, zero))

    lane = lax.iota(jnp.int32, _L)

    def allsum(x):
        for k in (1, 2, 4, 8):
            x = x + x.at[lane ^ k].get(mode="promise_in_bounds")
        return x

    with jax.named_scope("xchg"):
        pack = jnp.where(lane == 0, allsum(s),
                         jnp.where(lane == 1, allsum(ss), allsum(cnt)))
        pack_v[pl.ds(0, _L)] = pack
        pltpu.sync_copy(pack_v, shared.at[pl.ds(sid * _L, _L)])
        plsc.subcore_barrier()
        pltpu.sync_copy(shared, all_v)
        tot = all_v[pl.ds(0, _L)]
        for j in range(1, _NS):
            tot = tot + all_v[pl.ds(j * _L, _L)]

    n = jnp.maximum(jnp.full((_L,), tot[2]), 1.0)
    mean_v = jnp.full((_L,), tot[0]) / n
    var_v = jnp.maximum(jnp.full((_L,), tot[1]) / n - mean_v * mean_v,
                        0.0) + _EPS
    bits = lax.bitcast_convert_type(var_v, jnp.int32)
    y = lax.bitcast_convert_type(0x5F3759DF - (bits >> 1), jnp.float32)
    for _ in range(4):
        y = y * (1.5 - 0.5 * var_v * y * y)
    gbv = gb_v[pl.ds(0, _L)]
    scale_v = y * jnp.full((_L,), gbv[0])
    beta_v = jnp.full((_L,), gbv[8])

    coff = cid * _CHUNK

    def norm_step(t, _):
        x = nums_v[pl.ds(coff + t * _L, _L)]
        pres = x == x
        norm_v[pl.ds(t * _L, _L)] = (jnp.where(pres, x, 0.0)
                                     - mean_v) * scale_v + beta_v
        flags_v[pl.ds(t * _L, _L)] = jnp.where(
            pres, jnp.full((_L,), 1, jnp.int32), jnp.full((_L,), 0, jnp.int32))
        return 0

    with jax.named_scope("norm"):
        lax.fori_loop(0, _CHUNK // _L, norm_step, 0)

    outs = []
    for j in range(_NG):
        with jax.named_scope(f"blend{j}"):
            gathers[j].wait()

            def overwrite_group(g, _):
                off = j * _GCH + g * _L
                fvec = flags_v[pl.ds(off, _L)]
                nvec = norm_v[pl.ds(off, _L)]
                for l in range(_L):
                    @pl.when(fvec[l] > 0)
                    def _():
                        sp = jnp.full((_L,), nvec[l])
                        for c in range(_D // _L):
                            rows_v[off + l, pl.ds(c * _L, _L)] = sp
                return 0

            lax.fori_loop(0, _GCH // _L, overwrite_group, 0)
            outs.append(
                pltpu.async_copy(rows_v.at[pl.ds(j * _GCH, _GCH)],
                                 out_hbm.at[pl.ds(base + j * _GCH, _GCH)],
                                 osem))
    with jax.named_scope("odrain"):
        for cp in outs:
            cp.wait()


@jax.jit
def _run(values2d, numbers, emb_table, gamma, beta):
    mesh = plsc.VectorSubcoreMesh(core_axis_name="c", subcore_axis_name="s",
                                  num_cores=_NC, num_subcores=_NS)
    return pl.kernel(
        _sc_body,
        out_type=jax.ShapeDtypeStruct((_N, _D), jnp.float32),
        mesh=mesh,
        scratch_types=[
            pltpu.VMEM((_NG, _GCH), jnp.int32),
            pltpu.VMEM((_SLICE,), jnp.float32),
            pltpu.VMEM((_CHUNK, _D), jnp.float32),
            pltpu.VMEM((_CHUNK,), jnp.float32),
            pltpu.VMEM((_CHUNK,), jnp.int32),
            pltpu.VMEM((_L,), jnp.float32),
            pltpu.VMEM((_L,), jnp.float32),
            pltpu.VMEM((_NS * _L,), jnp.float32),
            pltpu.VMEM_SHARED((_NS * _L,), jnp.float32),
            pltpu.SemaphoreType.DMA((_NG,)),
            pltpu.SemaphoreType.DMA,
        ],
    )(values2d, numbers, emb_table, gamma, beta)


def kernel(values, numbers, emb_table, gamma, beta):
    values2d = values.astype(jnp.int32).reshape(_N // _GCH, _GCH)
    return _run(values2d, numbers.astype(jnp.float32), emb_table,
                gamma.astype(jnp.float32), beta.astype(jnp.float32))


# speedup vs baseline: 1.0167x; 1.0167x over previous
import functools

import jax
import jax.numpy as jnp
from jax import lax
from jax.experimental import pallas as pl
from jax.experimental.pallas import tpu as pltpu
from jax.experimental.pallas import tpu_sc as plsc

_EPS = 1e-5
_N, _V, _D = 16384, 100000, 128
_NC, _NS, _L = 2, 16, 16
_NW = _NC * _NS
_CHUNK = _N // _NW
_GCH = 64
_NG = _CHUNK // _GCH
_IDXROW = 128
_SLICE = _N // _NS


def _sc_body(values_hbm, numbers_hbm, table_hbm, gamma_hbm, beta_hbm, out_hbm,
             idx_v, nums_v, rows_v, norm_v, flags_v, gb_v, pack_v, all_v,
             shared, gsems, osem, ssem):
    cid = lax.axis_index("c")
    sid = lax.axis_index("s")
    wid = sid * _NC + cid
    base = wid * _CHUNK

    with jax.named_scope("stage"):
        nrows = _CHUNK // _IDXROW
        idx_cp = pltpu.async_copy(values_hbm.at[pl.ds(wid * nrows, nrows)],
                                  idx_v, ssem.at[0])
        num_cp = pltpu.async_copy(numbers_hbm.at[pl.ds(sid * _SLICE, _SLICE)],
                                  nums_v, ssem.at[1])
        g_cp = pltpu.async_copy(gamma_hbm, gb_v.at[pl.ds(0, 1)], ssem.at[1])
        b_cp = pltpu.async_copy(beta_hbm, gb_v.at[pl.ds(8, 1)], ssem.at[1])
        idx_cp.wait()
    with jax.named_scope("gfire"):
        gathers = [
            pltpu.async_copy(
                table_hbm.at[idx_v.at[j // 2, pl.ds((j % 2) * _GCH, _GCH)]],
                rows_v.at[pl.ds(j * _GCH, _GCH)], gsems.at[j])
            for j in range(_NG)
        ]
        num_cp.wait()
        g_cp.wait()
        b_cp.wait()

    def stats_step(i, carry):
        s, ss, cnt = carry
        for u in range(8):
            x = nums_v[pl.ds(i * 8 * _L + u * _L, _L)]
            pres = x == x
            xs = jnp.where(pres, x, 0.0)
            s = s + xs
            ss = ss + xs * xs
            cnt = cnt + jnp.where(pres, 1.0, 0.0)
        return s, ss, cnt

    with jax.named_scope("stats"):
        zero = jnp.zeros((_L,), jnp.float32)
        s, ss, cnt = lax.fori_loop(0, _SLICE // (8 * _L), stats_step,
                                   (zero, zero, zero))

    lane = lax.iota(jnp.int32, _L)

    def allsum(x):
        for k in (1, 2, 4, 8):
            x = x + x.at[lane ^ k].get(mode="promise_in_bounds")
        return x

    with jax.named_scope("xchg"):
        pack = jnp.where(lane == 0, allsum(s),
                         jnp.where(lane == 1, allsum(ss), allsum(cnt)))
        pack_v[pl.ds(0, _L)] = pack
        pltpu.sync_copy(pack_v, shared.at[pl.ds(sid * _L, _L)])
        plsc.subcore_barrier()
        pltpu.sync_copy(shared, all_v)
        tot = all_v[pl.ds(0, _L)]
        for j in range(1, _NS):
            tot = tot + all_v[pl.ds(j * _L, _L)]

    n = jnp.maximum(jnp.full((_L,), tot[2]), 1.0)
    mean_v = jnp.full((_L,), tot[0]) / n
    var_v = jnp.maximum(jnp.full((_L,), tot[1]) / n - mean_v * mean_v,
                        0.0) + _EPS
    bits = lax.bitcast_convert_type(var_v, jnp.int32)
    y = lax.bitcast_convert_type(0x5F3759DF - (bits >> 1), jnp.float32)
    for _ in range(4):
        y = y * (1.5 - 0.5 * var_v * y * y)
    gbv = gb_v[pl.ds(0, _L)]
    scale_v = y * jnp.full((_L,), gbv[0])
    beta_v = jnp.full((_L,), gbv[8])

    coff = cid * _CHUNK

    def norm_step(t, _):
        x = nums_v[pl.ds(coff + t * _L, _L)]
        pres = x == x
        norm_v[pl.ds(t * _L, _L)] = (jnp.where(pres, x, 0.0)
                                     - mean_v) * scale_v + beta_v
        flags_v[pl.ds(t * _L, _L)] = jnp.where(
            pres, jnp.full((_L,), 1, jnp.int32), jnp.full((_L,), 0, jnp.int32))
        return 0

    with jax.named_scope("norm"):
        lax.fori_loop(0, _CHUNK // _L, norm_step, 0)

    outs = []
    for j in range(_NG):
        with jax.named_scope(f"blend{j}"):
            gathers[j].wait()

            def overwrite_group(g, _):
                off = j * _GCH + g * _L
                fvec = flags_v[pl.ds(off, _L)]
                nvec = norm_v[pl.ds(off, _L)]
                for l in range(_L):
                    @pl.when(fvec[l] > 0)
                    def _():
                        sp = jnp.full((_L,), nvec[l])
                        for c in range(_D // _L):
                            rows_v[off + l, pl.ds(c * _L, _L)] = sp
                return 0

            lax.fori_loop(0, _GCH // _L, overwrite_group, 0)
            outs.append(
                pltpu.async_copy(rows_v.at[pl.ds(j * _GCH, _GCH)],
                                 out_hbm.at[pl.ds(base + j * _GCH, _GCH)],
                                 osem))
    with jax.named_scope("odrain"):
        for cp in outs:
            cp.wait()


@jax.jit
def _run(values2d, numbers, emb_table, gamma, beta):
    mesh = plsc.VectorSubcoreMesh(core_axis_name="c", subcore_axis_name="s",
                                  num_cores=_NC, num_subcores=_NS)
    return pl.kernel(
        _sc_body,
        out_type=jax.ShapeDtypeStruct((_N, _D), jnp.float32),
        mesh=mesh,
        scratch_types=[
            pltpu.VMEM((_CHUNK // _IDXROW, _IDXROW), jnp.int32),
            pltpu.VMEM((_SLICE,), jnp.float32),
            pltpu.VMEM((_CHUNK, _D), jnp.float32),
            pltpu.VMEM((_CHUNK,), jnp.float32),
            pltpu.VMEM((_CHUNK,), jnp.int32),
            pltpu.VMEM((_L,), jnp.float32),
            pltpu.VMEM((_L,), jnp.float32),
            pltpu.VMEM((_NS * _L,), jnp.float32),
            pltpu.VMEM_SHARED((_NS * _L,), jnp.float32),
            pltpu.SemaphoreType.DMA((_NG,)),
            pltpu.SemaphoreType.DMA,
            pltpu.SemaphoreType.DMA((2,)),
        ],
    )(values2d, numbers, emb_table, gamma, beta)


def kernel(values, numbers, emb_table, gamma, beta):
    values2d = values.astype(jnp.int32).reshape(_N // _IDXROW, _IDXROW)
    return _run(values2d, numbers.astype(jnp.float32), emb_table,
                gamma.astype(jnp.float32), beta.astype(jnp.float32))
